# Initial kernel scaffold; baseline (speedup 1.0000x reference)
import jax
import jax.numpy as jnp
from jax.experimental import pallas as pl


def kernel(output, target):
    raise NotImplementedError("write your pallas kernel here")


# speedup vs baseline: 23.3601x; 23.3601x over previous
import functools

import jax
import jax.numpy as jnp
import numpy as np
from jax.experimental import pallas as pl
from jax.experimental.pallas import tpu as pltpu

_NB, _NA, _NC, _NH, _NW = 32, 5, 20, 19, 19
_NPIX = _NH * _NW
_ANCH = np.array(
    [[1.3221, 1.73145], [3.19275, 4.00944], [5.05587, 8.09892],
     [9.47112, 4.84053], [11.2364, 10.0071]], dtype=np.float32)
_THRESH = 0.6
_OBJ_SCALE = 5.0


def _sigmoid(x):
    return 1.0 / (1.0 + jnp.exp(-x))


def _iou(px, py, pw, ph, qx, qy, qw, qh):
    p_x1 = px - pw * 0.5
    p_x2 = px + pw * 0.5
    p_y1 = py - ph * 0.5
    p_y2 = py + ph * 0.5
    q_x1 = qx - qw * 0.5
    q_x2 = qx + qw * 0.5
    q_y1 = qy - qh * 0.5
    q_y2 = qy + qh * 0.5
    mx = jnp.minimum(p_x1, q_x1)
    Mx = jnp.maximum(p_x2, q_x2)
    my = jnp.minimum(p_y1, q_y1)
    My = jnp.maximum(p_y2, q_y2)
    w1 = p_x2 - p_x1
    h1 = p_y2 - p_y1
    w2 = q_x2 - q_x1
    h2 = q_y2 - q_y1
    cw = w1 + w2 - (Mx - mx)
    ch = h1 + h2 - (My - my)
    inter = jnp.where((cw <= 0) | (ch <= 0), 0.0, cw * ch)
    union = w1 * h1 + w2 * h2 - inter
    return inter / union


def _sel_anchor(idx, col):
    v = jnp.full(jnp.shape(idx), float(_ANCH[0, col]), jnp.float32)
    for a in range(1, _NA):
        v = jnp.where(idx == a, float(_ANCH[a, col]), v)
    return v


def _body(tl_ref, ts_ref, o_ref, res_ref):
    b_id = pl.program_id(0)

    gx_s = ts_ref[0, :, 1:2] * float(_NW)
    gy_s = ts_ref[0, :, 2:3] * float(_NH)
    gw_s = ts_ref[0, :, 3:4] * float(_NW)
    gh_s = ts_ref[0, :, 4:5] * float(_NH)

    iota_s = jax.lax.broadcasted_iota(jnp.int32, (50, 1), 0)
    bad_s = jnp.where(ts_ref[0, :, 1:2] != 0.0, 50, iota_s)
    first_bad = jnp.min(bad_s)
    valid_s = iota_s < first_bad

    best_iou = jnp.full((50, 1), -1.0, jnp.float32)
    best_n = jnp.zeros((50, 1), jnp.int32)
    for a in range(_NA):
        awa = float(_ANCH[a, 0])
        aha = float(_ANCH[a, 1])
        inter = jnp.minimum(gw_s, awa) * jnp.minimum(gh_s, aha)
        union = gw_s * gh_s + awa * aha - inter
        iou_a = inter / union
        take = iou_a > best_iou
        best_n = jnp.where(take, a, best_n)
        best_iou = jnp.where(take, iou_a, best_iou)

    gi_s = gx_s.astype(jnp.int32)
    gj_s = gy_s.astype(jnp.int32)
    pixsel_s = gj_s * _NW + gi_s
    pidx_s = best_n * _NPIX + pixsel_s

    aw_sel = _sel_anchor(best_n, 0)
    ah_sel = _sel_anchor(best_n, 1)
    tx = gx_s - gi_s.astype(jnp.float32)
    ty = gy_s - gj_s.astype(jnp.float32)
    tw = jnp.log(gw_s / aw_sel)
    th = jnp.log(gh_s / ah_sel)

    gx_l = tl_ref[0, 1, :] * float(_NW)
    gy_l = tl_ref[0, 2, :] * float(_NH)
    gw_l = tl_ref[0, 3, :] * float(_NW)
    gh_l = tl_ref[0, 4, :] * float(_NH)
    best_iou_l = jnp.full((50,), -1.0, jnp.float32)
    best_n_l = jnp.zeros((50,), jnp.int32)
    for a in range(_NA):
        awa = float(_ANCH[a, 0])
        aha = float(_ANCH[a, 1])
        inter = jnp.minimum(gw_l, awa) * jnp.minimum(gh_l, aha)
        union = gw_l * gh_l + awa * aha - inter
        iou_a = inter / union
        take = iou_a > best_iou_l
        best_n_l = jnp.where(take, a, best_n_l)
        best_iou_l = jnp.where(take, iou_a, best_iou_l)
    pidx_l = (best_n_l * _NPIX
              + gy_l.astype(jnp.int32) * _NW + gx_l.astype(jnp.int32))
    iota_l = jax.lax.broadcasted_iota(jnp.int32, (50,), 0)
    valid_l = iota_l < first_bad

    r_i = jax.lax.broadcasted_iota(jnp.int32, (50, 50), 0)
    c_i = jax.lax.broadcasted_iota(jnp.int32, (50, 50), 1)
    conflict = jnp.any(
        (pidx_s == pidx_l[None, :]) & (c_i > r_i) & valid_l[None, :],
        axis=1, keepdims=True)
    last = valid_s & jnp.logical_not(conflict)
    lastf = last.astype(jnp.float32)

    pix_row = jax.lax.broadcasted_iota(jnp.int32, (1, _NPIX), 1)
    gcol = (pix_row % _NW).astype(jnp.float32)
    grow = (pix_row // _NW).astype(jnp.float32)

    valid_mask = valid_s.astype(jnp.float32)

    loss_dense = 0.0
    g_sx = jnp.zeros((50, 1), jnp.float32)
    g_sy = jnp.zeros((50, 1), jnp.float32)
    g_w = jnp.zeros((50, 1), jnp.float32)
    g_h = jnp.zeros((50, 1), jnp.float32)
    g_conf = jnp.zeros((50, 1), jnp.float32)
    g_noobj = jnp.zeros((50, 1), jnp.float32)
    g_lse = jnp.zeros((50, 1), jnp.float32)
    g_cls0 = jnp.zeros((50, 1), jnp.float32)

    for a in range(_NA):
        x_raw = o_ref[0, a, 0, :][None, :]
        y_raw = o_ref[0, a, 1, :][None, :]
        w_raw = o_ref[0, a, 2, :][None, :]
        h_raw = o_ref[0, a, 3, :][None, :]
        c_raw = o_ref[0, a, 4, :][None, :]
        sx = _sigmoid(x_raw)
        sy = _sigmoid(y_raw)
        conf = _sigmoid(c_raw)

        a_eff = (b_id * _NA + a) // _NB
        aw_c = _sel_anchor(a_eff, 0)
        ah_c = _sel_anchor(a_eff, 1)
        pw = jnp.exp(w_raw) * aw_c
        ph = jnp.exp(h_raw) * ah_c
        px = sx + gcol
        py = sy + grow

        iou2 = _iou(px, py, pw, ph, gx_s, gy_s, gw_s, gh_s)
        iou2 = iou2 * valid_mask
        cur = jnp.maximum(jnp.max(iou2, axis=0, keepdims=True), 0.0)
        noobj = jnp.where(cur > _THRESH, 0.0, 1.0)

        loss_dense += 0.5 * (
            jnp.sum((sx - 0.5) ** 2) + jnp.sum((sy - 0.5) ** 2)
            + jnp.sum(w_raw ** 2) + jnp.sum(h_raw ** 2)
            + jnp.sum(noobj * conf * conf))

        cls_ch = [o_ref[0, a, 5 + c, :][None, :] for c in range(_NC)]
        m = cls_ch[0]
        for c in range(1, _NC):
            m = jnp.maximum(m, cls_ch[c])
        s = jnp.zeros_like(m)
        for c in range(_NC):
            s = s + jnp.exp(cls_ch[c] - m)
        lse = jnp.log(s) + m

        oh = ((best_n == a) & (pixsel_s == pix_row)).astype(jnp.float32)

        def gath(arr, oh=oh):
            return jnp.sum(oh * arr, axis=1, keepdims=True)

        g_sx += gath(sx)
        g_sy += gath(sy)
        g_w += gath(w_raw)
        g_h += gath(h_raw)
        g_conf += gath(conf)
        g_noobj += gath(noobj)
        g_lse += gath(lse)
        g_cls0 += gath(cls_ch[0])

    g_aeff = (b_id * _NA + best_n) // _NB
    g_aw = _sel_anchor(g_aeff, 0)
    g_ah = _sel_anchor(g_aeff, 1)
    g_px = g_sx + gi_s.astype(jnp.float32)
    g_py = g_sy + gj_s.astype(jnp.float32)
    g_pw = jnp.exp(g_w) * g_aw
    g_ph = jnp.exp(g_h) * g_ah
    iou_t = _iou(g_px, g_py, g_pw, g_ph, gx_s, gy_s, gw_s, gh_s)

    d_coord = 0.5 * ((g_sx - tx) ** 2 + (g_sy - ty) ** 2
                     + (g_w - tw) ** 2 + (g_h - th) ** 2
                     - (g_sx - 0.5) ** 2 - (g_sy - 0.5) ** 2
                     - g_w ** 2 - g_h ** 2)
    d_conf = 0.5 * (_OBJ_SCALE * (g_conf - iou_t) ** 2
                    - g_noobj * g_conf * g_conf)
    d_cls = g_lse - g_cls0

    corr = jnp.sum(lastf * (d_coord + d_conf + d_cls))
    total = loss_dense + corr
    res_ref[0, 0, :] = jnp.full((128,), total, jnp.float32)


@jax.jit
def kernel(output, target):
    tgt = target.reshape(_NB, 50, 5)
    tgt_l = tgt.transpose(0, 2, 1)
    out = output.reshape(_NB, _NA, 5 + _NC, _NPIX)
    res = pl.pallas_call(
        _body,
        grid=(_NB,),
        in_specs=[
            pl.BlockSpec((1, 5, 50), lambda b: (b, 0, 0)),
            pl.BlockSpec((1, 50, 5), lambda b: (b, 0, 0)),
            pl.BlockSpec((1, _NA, 5 + _NC, _NPIX), lambda b: (b, 0, 0, 0)),
        ],
        out_specs=pl.BlockSpec((1, 1, 128), lambda b: (b, 0, 0)),
        out_shape=jax.ShapeDtypeStruct((_NB, 1, 128), jnp.float32),
    )(tgt_l, tgt, out)
    return jnp.sum(res[:, 0, 0])
